# baseline (speedup 1.0000x reference)
import functools
import math

import jax
import jax.numpy as jnp
import numpy as np
from jax import lax
from jax.experimental import pallas as pl
from jax.experimental.pallas import tpu as pltpu
from jax.experimental.pallas import tpu_sc as plsc


def _sinusoidal_pe(maxlen, dim):
    pos = jnp.arange(maxlen, dtype=jnp.float32)
    i = np.arange(dim)
    terms = jnp.asarray(1.0 / (10000.0 ** (2 * (i // 2) / dim)), dtype=jnp.float32)
    pe_val = pos[:, None] * terms[None, :]
    even = pe_val[:, 0::2]
    pe = jnp.zeros((maxlen, dim), dtype=jnp.float32)
    pe = pe.at[:, 0::2].set(jnp.sin(even))
    pe = pe.at[:, 1::2].set(jnp.cos(even))
    return pe


@functools.partial(jax.jit, static_argnames=("batch", "seq", "dim", "scale"))
def _sc_gather_pe(table, idx2d, pe, *, batch, seq, dim, scale):
    NW = 32
    CHUNK = seq // 2
    seqs_per_w = batch // NW
    groups_per_row = dim // 16

    mesh = plsc.VectorSubcoreMesh(core_axis_name="c", subcore_axis_name="s")

    @functools.partial(
        pl.kernel,
        mesh=mesh,
        compiler_params=pltpu.CompilerParams(use_tc_tiling_on_sc=False),
        out_type=jax.ShapeDtypeStruct((batch, seq, dim), jnp.float32),
        scratch_types=[
            pltpu.VMEM((2 * seqs_per_w, CHUNK), jnp.int32),
            pltpu.VMEM((seq, dim), jnp.float32),
            pltpu.VMEM((seq, dim), jnp.float32),
            pltpu.SemaphoreType.DMA,
        ],
    )
    def k(table_hbm, idx_hbm, pe_hbm, out_hbm, idx_v, rows_v, pe_v, sem):
        wid = lax.axis_index("s") * 2 + lax.axis_index("c")
        pltpu.sync_copy(idx_hbm.at[pl.ds(wid * 2 * seqs_per_w, 2 * seqs_per_w)], idx_v)
        pltpu.sync_copy(pe_hbm, pe_v)

        def seq_body(si, carry):
            cp0 = pltpu.async_copy(
                table_hbm.at[idx_v.at[2 * si]], rows_v.at[pl.ds(0, CHUNK)], sem
            )
            cp1 = pltpu.async_copy(
                table_hbm.at[idx_v.at[2 * si + 1]], rows_v.at[pl.ds(CHUNK, CHUNK)], sem
            )
            cp0.wait()
            cp1.wait()

            def row_body(r, carry2):
                for q in range(groups_per_row):
                    sl = pl.ds(q * 16, 16)
                    rows_v[r, sl] = rows_v[r, sl] * scale + pe_v[r, sl]
                return carry2

            lax.fori_loop(0, seq, row_body, 0)
            pltpu.sync_copy(rows_v, out_hbm.at[wid * seqs_per_w + si])
            return carry

        lax.fori_loop(0, seqs_per_w, seq_body, 0)

    return k(table, idx2d, pe)


def kernel(inp, table):
    B, S = inp.shape
    V, D = table.shape
    CHUNK = S // 2
    idx2d = inp.astype(jnp.int32).reshape(B * S // CHUNK, CHUNK)
    pe = _sinusoidal_pe(S, D)
    scale = 1.0 / math.sqrt(float(B))
    return _sc_gather_pe(table, idx2d, pe, batch=B, seq=S, dim=D, scale=scale)
